# baseline (speedup 1.0000x reference)
import functools

import jax
import jax.numpy as jnp
from jax import lax
from jax.experimental import pallas as pl
from jax.experimental.pallas import tpu as pltpu
from jax.experimental.pallas import tpu_sc as plsc

_NUM_EMB = 1024
_DIM = 64
_BLK = 512
_N = 36864


def _argmin_body(x_ref, emb_ref, idx_ref):
    xb = x_ref[:]
    emb = emb_ref[:]
    sim = jax.lax.dot_general(
        xb, emb, (((1,), (0,)), ((), ())), preferred_element_type=jnp.float32
    )
    x2 = jnp.sum(xb ** 2, axis=1, keepdims=True)
    e2 = jnp.sum(emb ** 2, axis=0, keepdims=True)
    dist = x2 + e2 - 2.0 * sim
    minval = jnp.min(dist, axis=1, keepdims=True)
    lanes = jax.lax.broadcasted_iota(jnp.int32, dist.shape, 1)
    idx_ref[:] = jnp.min(
        jnp.where(dist == minval, lanes, _NUM_EMB), axis=1, keepdims=True
    )


def _indices(flat, embeddings):
    return pl.pallas_call(
        _argmin_body,
        grid=(_N // _BLK,),
        in_specs=[
            pl.BlockSpec((_BLK, _DIM), lambda i: (i, 0)),
            pl.BlockSpec((_DIM, _NUM_EMB), lambda i: (0, 0)),
        ],
        out_specs=pl.BlockSpec((_BLK, 1), lambda i: (i, 0)),
        out_shape=jax.ShapeDtypeStruct((_N, 1), jnp.int32),
    )(flat, embeddings)


def _make_gather():
    info = plsc.get_sparse_core_info()
    nc, ns = info.num_cores, info.num_subcores
    nw = nc * ns
    b_per_w = _N // nw
    chunks = b_per_w // 128
    mesh = plsc.VectorSubcoreMesh(core_axis_name="c", subcore_axis_name="s")

    @functools.partial(
        pl.kernel,
        mesh=mesh,
        out_type=jax.ShapeDtypeStruct((_N, _DIM), jnp.float32),
        scratch_types=[
            pltpu.VMEM((b_per_w,), jnp.int32),
            pltpu.VMEM((b_per_w, _DIM), jnp.float32),
            pltpu.SemaphoreType.DMA,
        ],
        compiler_params=pltpu.CompilerParams(use_tc_tiling_on_sc=False),
    )
    def gather(table_hbm, idx_hbm, out_hbm, idx_v, rows_v, sem):
        wid = lax.axis_index("s") * nc + lax.axis_index("c")
        pltpu.sync_copy(idx_hbm.at[pl.ds(wid * b_per_w, b_per_w)], idx_v)
        copies = []
        for j in range(chunks):
            copies.append(
                pltpu.async_copy(
                    table_hbm.at[idx_v.at[pl.ds(j * 128, 128)]],
                    rows_v.at[pl.ds(j * 128, 128)],
                    sem,
                )
            )
        for c in copies:
            c.wait()
        pltpu.sync_copy(rows_v, out_hbm.at[pl.ds(wid * b_per_w, b_per_w)])

    return gather


_gather = _make_gather()


def kernel(x, embeddings):
    flat = x.reshape(-1, _DIM)
    idx = _indices(flat, embeddings).reshape(_N)
    table = embeddings.T
    out = _gather(table, idx)
    return out.reshape(x.shape)


# speedup vs baseline: 1.2670x; 1.2670x over previous
import functools

import jax
import jax.numpy as jnp
from jax import lax
from jax.experimental import pallas as pl
from jax.experimental.pallas import tpu as pltpu
from jax.experimental.pallas import tpu_sc as plsc

_NUM_EMB = 1024
_DIM = 64
_BLK = 512
_N = 36864


def _argmin_body(x_ref, emb_ref, idx_ref):
    xb = x_ref[:].reshape(8, _BLK, _DIM)
    emb = emb_ref[:]
    sim = jax.lax.dot_general(
        xb, emb, (((2,), (0,)), ((), ())), preferred_element_type=jnp.float32
    )
    x2 = jnp.sum(xb ** 2, axis=2, keepdims=True)
    e2 = jnp.sum(emb ** 2, axis=0).reshape(1, 1, _NUM_EMB)
    dist = x2 + e2 - 2.0 * sim
    minval = jnp.min(dist, axis=2, keepdims=True)
    lanes = jax.lax.broadcasted_iota(jnp.int32, dist.shape, 2)
    idx = jnp.min(jnp.where(dist == minval, lanes, _NUM_EMB), axis=2)
    idx_ref[:] = idx.reshape(1, 8, _BLK)


def _indices(flat, embeddings):
    return pl.pallas_call(
        _argmin_body,
        grid=(_N // (8 * _BLK),),
        in_specs=[
            pl.BlockSpec((8 * _BLK, _DIM), lambda i: (i, 0)),
            pl.BlockSpec((_DIM, _NUM_EMB), lambda i: (0, 0)),
        ],
        out_specs=pl.BlockSpec((1, 8, _BLK), lambda i: (i, 0, 0)),
        out_shape=jax.ShapeDtypeStruct((_N // (8 * _BLK), 8, _BLK), jnp.int32),
    )(flat, embeddings)


def _make_gather():
    info = plsc.get_sparse_core_info()
    nc, ns = info.num_cores, info.num_subcores
    nw = nc * ns
    b_per_w = _N // nw
    chunks = b_per_w // 128
    mesh = plsc.VectorSubcoreMesh(core_axis_name="c", subcore_axis_name="s")

    @functools.partial(
        pl.kernel,
        mesh=mesh,
        out_type=jax.ShapeDtypeStruct((_N, _DIM), jnp.float32),
        scratch_types=[
            pltpu.VMEM((b_per_w,), jnp.int32),
            pltpu.VMEM((b_per_w, _DIM), jnp.float32),
            pltpu.SemaphoreType.DMA,
        ],
        compiler_params=pltpu.CompilerParams(use_tc_tiling_on_sc=False),
    )
    def gather(table_hbm, idx_hbm, out_hbm, idx_v, rows_v, sem):
        wid = lax.axis_index("s") * nc + lax.axis_index("c")
        pltpu.sync_copy(idx_hbm.at[pl.ds(wid * b_per_w, b_per_w)], idx_v)
        copies = []
        for j in range(chunks):
            copies.append(
                pltpu.async_copy(
                    table_hbm.at[idx_v.at[pl.ds(j * 128, 128)]],
                    rows_v.at[pl.ds(j * 128, 128)],
                    sem,
                )
            )
        for c in copies:
            c.wait()
        pltpu.sync_copy(rows_v, out_hbm.at[pl.ds(wid * b_per_w, b_per_w)])

    return gather


_gather = _make_gather()


def kernel(x, embeddings):
    flat = x.reshape(-1, _DIM)
    idx = _indices(flat, embeddings).reshape(_N)
    table = embeddings.T
    out = _gather(table, idx)
    return out.reshape(x.shape)


# speedup vs baseline: 1.2971x; 1.0237x over previous
import functools

import jax
import jax.numpy as jnp
from jax import lax
from jax.experimental import pallas as pl
from jax.experimental.pallas import tpu as pltpu
from jax.experimental.pallas import tpu_sc as plsc

_NUM_EMB = 1024
_DIM = 64
_BLK = 512
_N = 36864


def _argmin_body(x_ref, emb_ref, idx_ref):
    xb = x_ref[:].reshape(8, _BLK, _DIM)
    emb = emb_ref[:]
    sim = jax.lax.dot_general(
        xb, emb, (((2,), (0,)), ((), ())), preferred_element_type=jnp.float32
    )
    x2 = jnp.sum(xb ** 2, axis=2, keepdims=True)
    e2 = jnp.sum(emb ** 2, axis=0).reshape(1, 1, _NUM_EMB)
    dist = x2 + e2 - 2.0 * sim
    minval = jnp.min(dist, axis=2, keepdims=True)
    lanes = jax.lax.broadcasted_iota(jnp.int32, dist.shape, 2)
    idx = jnp.min(jnp.where(dist == minval, lanes, _NUM_EMB), axis=2)
    idx_ref[:] = idx.reshape(1, 8, _BLK)


def _indices(flat, embeddings):
    return pl.pallas_call(
        _argmin_body,
        grid=(_N // (8 * _BLK),),
        in_specs=[
            pl.BlockSpec((8 * _BLK, _DIM), lambda i: (i, 0)),
            pl.BlockSpec((_DIM, _NUM_EMB), lambda i: (0, 0)),
        ],
        out_specs=pl.BlockSpec((1, 8, _BLK), lambda i: (i, 0, 0)),
        out_shape=jax.ShapeDtypeStruct((_N // (8 * _BLK), 8, _BLK), jnp.int32),
    )(flat, embeddings)


def _make_gather():
    info = plsc.get_sparse_core_info()
    nc, ns = info.num_cores, info.num_subcores
    nw = nc * ns
    b_per_w = _N // nw
    chunks = b_per_w // 128
    mesh = plsc.VectorSubcoreMesh(core_axis_name="c", subcore_axis_name="s")

    @functools.partial(
        pl.kernel,
        mesh=mesh,
        out_type=jax.ShapeDtypeStruct((_N, 2 * _DIM), jnp.float32),
        scratch_types=[
            pltpu.VMEM((b_per_w,), jnp.int32),
            pltpu.VMEM((640, 2 * _DIM), jnp.float32),
            pltpu.SemaphoreType.DMA,
        ],
    )
    def gather(table_hbm, idx_hbm, out_hbm, idx_v, rows_v, sem):
        wid = lax.axis_index("s") * nc + lax.axis_index("c")
        base = wid * b_per_w
        pltpu.sync_copy(idx_hbm.at[pl.ds(base, b_per_w)], idx_v)
        for lo, n in ((0, 4), (4, 5)):
            copies = []
            for j in range(n):
                copies.append(
                    pltpu.async_copy(
                        table_hbm.at[idx_v.at[pl.ds((lo + j) * 128, 128)]],
                        rows_v.at[pl.ds(j * 128, 128)],
                        sem,
                    )
                )
            for c in copies:
                c.wait()
            pltpu.sync_copy(
                rows_v.at[pl.ds(0, n * 128)],
                out_hbm.at[pl.ds(base + lo * 128, n * 128)],
            )

    return gather


_gather = _make_gather()


def kernel(x, embeddings):
    flat = x.reshape(-1, _DIM)
    idx = _indices(flat, embeddings).reshape(_N)
    table = jnp.pad(embeddings.T, ((0, 0), (0, _DIM)))
    out = _gather(table, idx)[:, :_DIM]
    return out.reshape(x.shape)
